# baseline (speedup 1.0000x reference)
import functools

import jax
import jax.numpy as jnp
from jax import lax
from jax.experimental import pallas as pl
from jax.experimental.pallas import tpu as pltpu
from jax.experimental.pallas import tpu_sc as plsc

_CHUNK = 32
_NBUF = 2


def kernel(input_seq, weights):
    batch, seq_len = input_seq.shape
    _, emb = weights.shape

    info = plsc.get_sparse_core_info()
    num_workers = info.num_cores * info.num_subcores
    rows_per_w = seq_len // num_workers
    n_chunks = rows_per_w // _CHUNK

    mesh = plsc.VectorSubcoreMesh(core_axis_name="c", subcore_axis_name="s")

    @functools.partial(
        pl.kernel,
        out_type=jax.ShapeDtypeStruct((batch, seq_len, emb), weights.dtype),
        mesh=mesh,
        scratch_types=[
            pltpu.VMEM((_NBUF, _CHUNK, emb), jnp.float32),
            pltpu.SemaphoreType.DMA,
            pltpu.SemaphoreType.DMA,
        ],
    )
    def _bcast(w_hbm, out_hbm, buf, gsem, ssem):
        wid = lax.axis_index("s") * info.num_cores + lax.axis_index("c")
        base = wid * rows_per_w

        def gather(ci):
            return pltpu.make_async_copy(
                w_hbm.at[pl.ds(base + ci * _CHUNK, _CHUNK)],
                buf.at[ci % _NBUF],
                gsem,
            )

        def scatters(ci):
            return [
                pltpu.make_async_copy(
                    buf.at[ci % _NBUF],
                    out_hbm.at[b, pl.ds(base + ci * _CHUNK, _CHUNK)],
                    ssem,
                )
                for b in range(batch)
            ]

        gather(0).start()
        for ci in range(n_chunks):
            if ci + 1 < n_chunks:
                if ci + 1 >= _NBUF:
                    for c in scatters(ci + 1 - _NBUF):
                        c.wait()
                gather(ci + 1).start()
            gather(ci).wait()
            for c in scatters(ci):
                c.start()
        for ci in range(max(0, n_chunks - _NBUF), n_chunks):
            for c in scatters(ci):
                c.wait()

    return _bcast(weights)
